# Initial kernel scaffold; baseline (speedup 1.0000x reference)
import jax
import jax.numpy as jnp
from jax.experimental import pallas as pl


def kernel(idx, targets, table):
    raise NotImplementedError("write your pallas kernel here")


# speedup vs baseline: 1.3693x; 1.3693x over previous
import functools

import jax
import jax.numpy as jnp
from jax import lax
from jax.experimental import pallas as pl
from jax.experimental.pallas import tpu as pltpu
from jax.experimental.pallas import tpu_sc as plsc


def _lse_body(tab_ref, out_ref):
    x = tab_ref[...]
    m = jnp.max(x, axis=1, keepdims=True)
    s = jnp.sum(jnp.exp(x - m), axis=1, keepdims=True)
    out_ref[...] = m + jnp.log(s)


def _finish_body(part_ref, out_ref, *, inv_n):
    out_ref[...] = jnp.sum(part_ref[...], keepdims=True).reshape(1, 1) * inv_n


def _make_sc_gather(n_tok, vocab, nc, ns):
    nw = nc * ns
    per_w = n_tok // nw
    ck = 64
    n_chunks = per_w // ck
    mesh = plsc.VectorSubcoreMesh(core_axis_name="c", subcore_axis_name="s")

    @functools.partial(
        pl.kernel,
        out_type=(
            jax.ShapeDtypeStruct((n_tok, vocab), jnp.float32),
            jax.ShapeDtypeStruct((nw * 16,), jnp.float32),
        ),
        mesh=mesh,
        compiler_params=pltpu.CompilerParams(
            use_tc_tiling_on_sc=False, needs_layout_passes=False
        ),
        scratch_types=[
            pltpu.VMEM((ck,), jnp.int32),
            pltpu.VMEM((ck,), jnp.int32),
            pltpu.VMEM((ck, vocab), jnp.float32),
            pltpu.VMEM((vocab,), jnp.float32),
            pltpu.VMEM((16,), jnp.float32),
            pltpu.VMEM((16,), jnp.float32),
            pltpu.VMEM((16,), jnp.float32),
            pltpu.SemaphoreType.DMA,
        ],
    )
    def sc_k(table_hbm, idx_hbm, tgt_hbm, lse_hbm, out_hbm, part_hbm,
             idx_v, tgt_v, rows_v, lse_v, p_acc, l_acc, stage_v, sem):
        c = lax.axis_index("c")
        s = lax.axis_index("s")
        wid = s * nc + c
        base0 = wid * per_w
        pltpu.sync_copy(lse_hbm, lse_v)
        p_acc[...] = jnp.zeros((16,), jnp.float32)
        l_acc[...] = jnp.zeros((16,), jnp.float32)

        def chunk(i, carry):
            base = base0 + i * ck
            pltpu.sync_copy(idx_hbm.at[pl.ds(base, ck)], idx_v)
            pltpu.sync_copy(tgt_hbm.at[pl.ds(base, ck)], tgt_v)
            pltpu.async_copy(table_hbm.at[idx_v], rows_v, sem).wait()
            pltpu.sync_copy(rows_v, out_hbm.at[pl.ds(base, ck)])
            lane = lax.iota(jnp.int32, 16)
            for j in range(ck // 16):
                rid = lane + (16 * j)
                tg = tgt_v[pl.ds(16 * j, 16)]
                ids = idx_v[pl.ds(16 * j, 16)]
                p_acc[...] = p_acc[...] + plsc.load_gather(rows_v, [rid, tg])
                l_acc[...] = l_acc[...] + plsc.load_gather(lse_v, [ids])
            return carry

        lax.fori_loop(0, n_chunks, chunk, 0)
        stage_v[...] = l_acc[...] - p_acc[...]
        pltpu.sync_copy(stage_v, part_hbm.at[pl.ds(wid * 16, 16)])

    return sc_k


def kernel(idx, targets, table):
    b, t = idx.shape
    v, c = table.shape
    n = b * t
    idx_f = idx.reshape(n).astype(jnp.int32)
    tgt_f = targets.reshape(n).astype(jnp.int32)

    lse = pl.pallas_call(
        _lse_body,
        out_shape=jax.ShapeDtypeStruct((v, 1), jnp.float32),
    )(table).reshape(v)

    info = plsc.get_sparse_core_info()
    sc_k = _make_sc_gather(n, c, info.num_cores, info.num_subcores)
    logits_flat, parts = sc_k(table, idx_f, tgt_f, lse)

    loss = pl.pallas_call(
        functools.partial(_finish_body, inv_n=1.0 / float(n)),
        out_shape=jax.ShapeDtypeStruct((1, 1), jnp.float32),
    )(parts.reshape(info.num_cores * info.num_subcores, 16))[0, 0]

    return logits_flat.reshape(b, t, c), loss


# speedup vs baseline: 1.8929x; 1.3823x over previous
import functools

import jax
import jax.numpy as jnp
from jax import lax
from jax.experimental import pallas as pl
from jax.experimental.pallas import tpu as pltpu
from jax.experimental.pallas import tpu_sc as plsc


def _lse_body(tab_ref, out_ref):
    x = tab_ref[...]
    m = jnp.max(x, axis=1, keepdims=True)
    s = jnp.sum(jnp.exp(x - m), axis=1, keepdims=True)
    out_ref[...] = m + jnp.log(s)


def _finish_body(part_ref, out_ref, *, inv_n):
    out_ref[...] = jnp.sum(part_ref[...], keepdims=True).reshape(1, 1) * inv_n


def _gather_body(idx_sref, tab_ref, out_ref, *, bb, tt):
    i = pl.program_id(0)
    base = i * bb * tt

    def row(b_loc, _):
        def tok(t_loc, carry):
            sidx = idx_sref[base + b_loc * tt + t_loc]
            out_ref[b_loc, pl.ds(t_loc, 1), :] = tab_ref[pl.ds(sidx, 1), :]
            return carry

        return lax.fori_loop(0, tt, tok, _, unroll=10)

    lax.fori_loop(0, bb, row, 0)


def _make_sc_loss(n_tok, vocab, nc, ns):
    nw = nc * ns
    per_w = n_tok // nw
    ck = 64
    n_chunks = per_w // ck
    mesh = plsc.VectorSubcoreMesh(core_axis_name="c", subcore_axis_name="s")

    @functools.partial(
        pl.kernel,
        out_type=jax.ShapeDtypeStruct((nw * 16,), jnp.float32),
        mesh=mesh,
        compiler_params=pltpu.CompilerParams(
            use_tc_tiling_on_sc=False, needs_layout_passes=False
        ),
        scratch_types=[
            pltpu.VMEM((ck,), jnp.int32),
            pltpu.VMEM((ck,), jnp.int32),
            pltpu.VMEM((ck, vocab), jnp.float32),
            pltpu.VMEM((vocab,), jnp.float32),
            pltpu.VMEM((16,), jnp.float32),
            pltpu.VMEM((16,), jnp.float32),
            pltpu.VMEM((16,), jnp.float32),
            pltpu.SemaphoreType.DMA,
        ],
    )
    def sc_k(table_hbm, idx_hbm, tgt_hbm, lse_hbm, part_hbm,
             idx_v, tgt_v, rows_v, lse_v, p_acc, l_acc, stage_v, sem):
        c = lax.axis_index("c")
        s = lax.axis_index("s")
        wid = s * nc + c
        base0 = wid * per_w
        pltpu.sync_copy(lse_hbm, lse_v)
        p_acc[...] = jnp.zeros((16,), jnp.float32)
        l_acc[...] = jnp.zeros((16,), jnp.float32)

        def chunk(i, carry):
            base = base0 + i * ck
            pltpu.sync_copy(idx_hbm.at[pl.ds(base, ck)], idx_v)
            pltpu.sync_copy(tgt_hbm.at[pl.ds(base, ck)], tgt_v)
            pltpu.async_copy(table_hbm.at[idx_v], rows_v, sem).wait()
            lane = lax.iota(jnp.int32, 16)
            for j in range(ck // 16):
                rid = lane + (16 * j)
                tg = tgt_v[pl.ds(16 * j, 16)]
                ids = idx_v[pl.ds(16 * j, 16)]
                p_acc[...] = p_acc[...] + plsc.load_gather(rows_v, [rid, tg])
                l_acc[...] = l_acc[...] + plsc.load_gather(lse_v, [ids])
            return carry

        lax.fori_loop(0, n_chunks, chunk, 0)
        stage_v[...] = l_acc[...] - p_acc[...]
        pltpu.sync_copy(stage_v, part_hbm.at[pl.ds(wid * 16, 16)])

    return sc_k


def kernel(idx, targets, table):
    b, t = idx.shape
    v, c = table.shape
    n = b * t
    idx_f = idx.reshape(n).astype(jnp.int32)
    tgt_f = targets.reshape(n).astype(jnp.int32)

    lse = pl.pallas_call(
        _lse_body,
        out_shape=jax.ShapeDtypeStruct((v, 1), jnp.float32),
    )(table).reshape(v)

    info = plsc.get_sparse_core_info()
    nw = info.num_cores * info.num_subcores
    sc_k = _make_sc_loss(n, c, info.num_cores, info.num_subcores)
    parts = sc_k(table, idx_f, tgt_f, lse)

    bb = 8
    grid_spec = pltpu.PrefetchScalarGridSpec(
        num_scalar_prefetch=1,
        grid=(b // bb,),
        in_specs=[pl.BlockSpec((v, c), lambda i, idx_ref: (0, 0))],
        out_specs=pl.BlockSpec((bb, t, c), lambda i, idx_ref: (i, 0, 0)),
    )
    logits = pl.pallas_call(
        functools.partial(_gather_body, bb=bb, tt=t),
        grid_spec=grid_spec,
        out_shape=jax.ShapeDtypeStruct((b, t, c), jnp.float32),
    )(idx_f, table)

    loss = pl.pallas_call(
        functools.partial(_finish_body, inv_n=1.0 / float(n)),
        out_shape=jax.ShapeDtypeStruct((1, 1), jnp.float32),
    )(parts.reshape(nw, 16))[0, 0]

    return logits, loss
